# baseline (speedup 1.0000x reference)
import functools

import jax
import jax.numpy as jnp
from jax import lax
from jax.experimental import pallas as pl
from jax.experimental.pallas import tpu as pltpu
from jax.experimental.pallas import tpu_sc as plsc

_LANES = 16
_ROWS_PER_CHUNK = 20
_UNROLL = 8
_SC_ROWS = 20000
_TC_BLOCK = 400


def _make_sc_packer(dim, row_base, sc_rows, num_workers, num_cores):
    rows = _ROWS_PER_CHUNK
    rows_per_worker = sc_rows // num_workers
    chunks = (rows_per_worker + rows - 1) // rows
    chunk_elems = rows * dim
    main_chunks = (chunks - 1) if (chunks % 2) else chunks

    mesh = plsc.VectorSubcoreMesh(core_axis_name="c", subcore_axis_name="s")

    @functools.partial(
        pl.kernel,
        mesh=mesh,
        out_type=jax.ShapeDtypeStruct((sc_rows * dim,), jnp.float32),
        scratch_types=[
            pltpu.VMEM((chunk_elems,), jnp.int32),
            pltpu.VMEM((chunk_elems + _LANES,), jnp.float32),
            pltpu.VMEM((chunk_elems + _LANES,), jnp.float32),
            pltpu.VMEM((chunk_elems,), jnp.float32),
            pltpu.VMEM((chunk_elems,), jnp.float32),
            pltpu.SemaphoreType.DMA,
            pltpu.SemaphoreType.DMA,
            pltpu.SemaphoreType.DMA,
            pltpu.SemaphoreType.DMA,
        ],
        compiler_params=pltpu.CompilerParams(needs_layout_passes=False),
    )
    def packer(x_hbm, idx_hbm, out_hbm, idx_v, ib0, ib1, ob0, ob1,
               isem0, isem1, osem0, osem1):
        ibufs, obufs = (ib0, ib1), (ob0, ob1)
        isems, osems = (isem0, isem1), (osem0, osem1)
        wid = lax.axis_index("s") * num_cores + lax.axis_index("c")
        base = wid * rows_per_worker * dim
        last_off = (rows_per_worker - rows) * dim

        def chunk_off(ci):
            return base + jnp.minimum(ci * chunk_elems, last_off)

        def start_in(ci, b):
            src = x_hbm.at[pl.ds(row_base * dim + chunk_off(ci), chunk_elems)]
            pltpu.async_copy(src, ibufs[b].at[pl.ds(0, chunk_elems)], isems[b])

        def wait_in(b):
            pltpu.make_async_copy(x_hbm.at[pl.ds(0, chunk_elems)],
                                  ibufs[b].at[pl.ds(0, chunk_elems)],
                                  isems[b]).wait()

        def start_out(ci, b):
            pltpu.async_copy(obufs[b],
                             out_hbm.at[pl.ds(chunk_off(ci), chunk_elems)],
                             osems[b])

        def wait_out(b):
            pltpu.make_async_copy(obufs[b],
                                  out_hbm.at[pl.ds(0, chunk_elems)],
                                  osems[b]).wait()

        def compute(b):
            ib, ob = ibufs[b], obufs[b]

            @plsc.parallel_loop(0, chunk_elems, step=_LANES, unroll=_UNROLL)
            def _(s):
                idx = idx_v[pl.ds(s, _LANES)]
                ob[pl.ds(s, _LANES)] = plsc.load_gather(ib, [idx])

        pltpu.sync_copy(idx_hbm, idx_v)
        zeros = jnp.zeros((_LANES,), jnp.float32)
        ib0[pl.ds(chunk_elems, _LANES)] = zeros
        ib1[pl.ds(chunk_elems, _LANES)] = zeros
        start_in(0, 0)
        start_in(1, 1)

        @pl.loop(0, main_chunks, step=2)
        def _(i):
            for b in range(2):
                ci = i + b
                wait_in(b)

                @pl.when(ci >= 2)
                def _():
                    wait_out(b)

                compute(b)
                start_out(ci, b)

                @pl.when(ci + 2 < chunks)
                def _():
                    start_in(ci + 2, b)

        if main_chunks != chunks:
            wait_in(0)
            wait_out(0)
            compute(0)
            start_out(chunks - 1, 0)
        wait_out(0)
        wait_out(1)

    return packer


def _tc_body(x_ref, idx_ref, msk_ref, o_ref):
    blk, dim = x_ref.shape
    groups = dim // 128
    x = x_ref[...]
    idx = idx_ref[...]
    lidx = jnp.bitwise_and(idx, 127)
    sgrp = jnp.right_shift(idx, 7)
    for j in range(groups):
        lo, hi = j * 128, (j + 1) * 128
        idxb = jnp.broadcast_to(lidx[:, lo:hi], (blk, 128))
        sg = sgrp[:, lo:hi]
        acc = jnp.zeros((blk, 128), jnp.float32)
        for s in range(groups):
            g = jnp.take_along_axis(x[:, s * 128:(s + 1) * 128], idxb,
                                    axis=1, mode="promise_in_bounds")
            acc = jnp.where(sg == s, g, acc)
        o_ref[:, lo:hi] = jnp.where(msk_ref[:, lo:hi] != 0,
                                    jnp.float32(0.0), acc)


def _make_tc_packer(n_rows, tc_rows, dim):
    blk = _TC_BLOCK
    return pl.pallas_call(
        _tc_body,
        grid=(tc_rows // blk,),
        in_specs=[
            pl.BlockSpec((blk, dim), lambda i: (i, 0)),
            pl.BlockSpec((1, dim), lambda i: (0, 0)),
            pl.BlockSpec((1, dim), lambda i: (0, 0)),
        ],
        out_specs=pl.BlockSpec((blk, dim), lambda i: (i, 0)),
        out_shape=jax.ShapeDtypeStruct((n_rows, dim), jnp.float32),
    )


def kernel(x_flat, pack_index, pad_mask):
    n, dim = x_flat.shape
    num_coeffs, num_channels = pad_mask.shape[1], pad_mask.shape[2]
    info = plsc.get_sparse_core_info()
    num_workers = info.num_cores * info.num_subcores
    sc_rows = _SC_ROWS
    tc_rows = n - sc_rows
    assert sc_rows % num_workers == 0
    assert sc_rows // num_workers >= _ROWS_PER_CHUNK
    assert tc_rows % _TC_BLOCK == 0

    pidx = pack_index.astype(jnp.int32)
    mask_flat = pad_mask.reshape(-1)

    chunk_elems = _ROWS_PER_CHUNK * dim
    offs = jnp.arange(_ROWS_PER_CHUNK, dtype=jnp.int32)[:, None] * dim
    idx_full = jnp.where(mask_flat[None, :], chunk_elems,
                         pidx[None, :] + offs).reshape(-1)
    sc_packer = _make_sc_packer(dim, tc_rows, sc_rows, num_workers,
                                info.num_cores)
    sc_out = sc_packer(x_flat.reshape(-1), idx_full)

    tc_packer = _make_tc_packer(n, tc_rows, dim)
    tc_out = tc_packer(x_flat, pidx[None, :],
                       mask_flat.astype(jnp.int32)[None, :])

    out = lax.dynamic_update_slice(
        tc_out.reshape(n, num_coeffs, num_channels),
        sc_out.reshape(sc_rows, num_coeffs, num_channels),
        (tc_rows, 0, 0))
    return out


# speedup vs baseline: 1.0292x; 1.0292x over previous
import functools

import jax
import jax.numpy as jnp
from jax import lax
from jax.experimental import pallas as pl
from jax.experimental.pallas import tpu as pltpu
from jax.experimental.pallas import tpu_sc as plsc

_LANES = 16
_ROWS_PER_CHUNK = 20
_UNROLL = 8
_SC_ROWS = 20000
_TC_BLOCK = 400


def _make_sc_packer(dim, row_base, sc_rows, num_workers, num_cores):
    rows = _ROWS_PER_CHUNK
    rows_per_worker = sc_rows // num_workers
    chunks = (rows_per_worker + rows - 1) // rows
    chunk_elems = rows * dim
    main_chunks = (chunks - 1) if (chunks % 2) else chunks

    mesh = plsc.VectorSubcoreMesh(core_axis_name="c", subcore_axis_name="s")

    @functools.partial(
        pl.kernel,
        mesh=mesh,
        out_type=jax.ShapeDtypeStruct((sc_rows * dim,), jnp.float32),
        scratch_types=[
            pltpu.VMEM((chunk_elems,), jnp.int32),
            pltpu.VMEM((chunk_elems + _LANES,), jnp.float32),
            pltpu.VMEM((chunk_elems + _LANES,), jnp.float32),
            pltpu.VMEM((chunk_elems,), jnp.float32),
            pltpu.VMEM((chunk_elems,), jnp.float32),
            pltpu.SemaphoreType.DMA,
            pltpu.SemaphoreType.DMA,
            pltpu.SemaphoreType.DMA,
            pltpu.SemaphoreType.DMA,
        ],
        compiler_params=pltpu.CompilerParams(needs_layout_passes=False),
    )
    def packer(x_hbm, idx_hbm, out_hbm, idx_v, ib0, ib1, ob0, ob1,
               isem0, isem1, osem0, osem1):
        ibufs, obufs = (ib0, ib1), (ob0, ob1)
        isems, osems = (isem0, isem1), (osem0, osem1)
        wid = lax.axis_index("s") * num_cores + lax.axis_index("c")
        base = wid * rows_per_worker * dim
        last_off = (rows_per_worker - rows) * dim

        def chunk_off(ci):
            return base + jnp.minimum(ci * chunk_elems, last_off)

        def start_in(ci, b):
            src = x_hbm.at[pl.ds(row_base * dim + chunk_off(ci), chunk_elems)]
            pltpu.async_copy(src, ibufs[b].at[pl.ds(0, chunk_elems)], isems[b])

        def wait_in(b):
            pltpu.make_async_copy(x_hbm.at[pl.ds(0, chunk_elems)],
                                  ibufs[b].at[pl.ds(0, chunk_elems)],
                                  isems[b]).wait()

        def start_out(ci, b):
            pltpu.async_copy(obufs[b],
                             out_hbm.at[pl.ds(chunk_off(ci), chunk_elems)],
                             osems[b])

        def wait_out(b):
            pltpu.make_async_copy(obufs[b],
                                  out_hbm.at[pl.ds(0, chunk_elems)],
                                  osems[b]).wait()

        def compute(b):
            ib, ob = ibufs[b], obufs[b]

            @plsc.parallel_loop(0, chunk_elems, step=_LANES, unroll=_UNROLL)
            def _(s):
                idx = idx_v[pl.ds(s, _LANES)]
                ob[pl.ds(s, _LANES)] = plsc.load_gather(ib, [idx])

        pltpu.sync_copy(idx_hbm, idx_v)
        zeros = jnp.zeros((_LANES,), jnp.float32)
        ib0[pl.ds(chunk_elems, _LANES)] = zeros
        ib1[pl.ds(chunk_elems, _LANES)] = zeros
        start_in(0, 0)
        start_in(1, 1)

        @pl.loop(0, main_chunks, step=2)
        def _(i):
            for b in range(2):
                ci = i + b
                wait_in(b)

                @pl.when(ci >= 2)
                def _():
                    wait_out(b)

                compute(b)
                start_out(ci, b)

                @pl.when(ci + 2 < chunks)
                def _():
                    start_in(ci + 2, b)

        if main_chunks != chunks:
            wait_in(0)
            wait_out(0)
            compute(0)
            start_out(chunks - 1, 0)
        wait_out(0)
        wait_out(1)

    return packer


def _tc_body(x_ref, idx_ref, msk_ref, o_ref):
    blk, dim = x_ref.shape
    groups = dim // 128
    x = x_ref[...]
    idx = idx_ref[...]
    lidx = jnp.bitwise_and(idx, 127)
    sgrp = jnp.right_shift(idx, 7)
    for j in range(groups):
        lo, hi = j * 128, (j + 1) * 128
        idxb = jnp.broadcast_to(lidx[:, lo:hi], (blk, 128))
        sg = sgrp[:, lo:hi]
        acc = jnp.zeros((blk, 128), jnp.float32)
        for s in range(groups):
            g = jnp.take_along_axis(x[:, s * 128:(s + 1) * 128], idxb,
                                    axis=1, mode="promise_in_bounds")
            acc = jnp.where(sg == s, g, acc)
        o_ref[:, j, :] = jnp.where(msk_ref[:, j, :] != 0,
                                   jnp.float32(0.0), acc)


def _make_tc_packer(n_rows, tc_rows, dim, num_coeffs, num_channels):
    blk = _TC_BLOCK
    return pl.pallas_call(
        _tc_body,
        grid=(tc_rows // blk,),
        in_specs=[
            pl.BlockSpec((blk, dim), lambda i: (i, 0)),
            pl.BlockSpec((1, dim), lambda i: (0, 0)),
            pl.BlockSpec((1, num_coeffs, num_channels), lambda i: (0, 0, 0)),
        ],
        out_specs=pl.BlockSpec((blk, num_coeffs, num_channels),
                               lambda i: (i, 0, 0)),
        out_shape=jax.ShapeDtypeStruct((n_rows, num_coeffs, num_channels),
                                       jnp.float32),
    )


def kernel(x_flat, pack_index, pad_mask):
    n, dim = x_flat.shape
    num_coeffs, num_channels = pad_mask.shape[1], pad_mask.shape[2]
    info = plsc.get_sparse_core_info()
    num_workers = info.num_cores * info.num_subcores
    sc_rows = _SC_ROWS
    tc_rows = n - sc_rows
    assert sc_rows % num_workers == 0
    assert sc_rows // num_workers >= _ROWS_PER_CHUNK
    assert tc_rows % _TC_BLOCK == 0

    pidx = pack_index.astype(jnp.int32)
    mask_flat = pad_mask.reshape(-1)

    chunk_elems = _ROWS_PER_CHUNK * dim
    offs = jnp.arange(_ROWS_PER_CHUNK, dtype=jnp.int32)[:, None] * dim
    idx_full = jnp.where(mask_flat[None, :], chunk_elems,
                         pidx[None, :] + offs).reshape(-1)
    sc_packer = _make_sc_packer(dim, tc_rows, sc_rows, num_workers,
                                info.num_cores)
    sc_out = sc_packer(x_flat.reshape(-1), idx_full)

    tc_packer = _make_tc_packer(n, tc_rows, dim, num_coeffs, num_channels)
    tc_out = tc_packer(x_flat, pidx[None, :], pad_mask.astype(jnp.int32))

    out = lax.dynamic_update_slice(
        tc_out,
        sc_out.reshape(sc_rows, num_coeffs, num_channels),
        (tc_rows, 0, 0))
    return out


# speedup vs baseline: 2.4663x; 2.3962x over previous
import functools

import jax
import jax.numpy as jnp
from jax import lax
from jax.experimental import pallas as pl
from jax.experimental.pallas import tpu as pltpu
from jax.experimental.pallas import tpu_sc as plsc

_LANES = 16
_ROWS_PER_CHUNK = 16
_SC_ROWS = 25600
_TC_BLOCK = 400


def _make_sc_packer(dim, row_base, sc_rows, num_workers, num_cores,
                    num_coeffs, num_channels):
    rows = _ROWS_PER_CHUNK
    rows_per_worker = sc_rows // num_workers
    chunks = rows_per_worker // rows
    assert chunks % 2 == 0 and chunks * rows == rows_per_worker

    mesh = plsc.VectorSubcoreMesh(core_axis_name="c", subcore_axis_name="s")

    @functools.partial(
        pl.kernel,
        mesh=mesh,
        out_type=jax.ShapeDtypeStruct((num_coeffs, sc_rows, num_channels),
                                      jnp.float32),
        scratch_types=[
            pltpu.VMEM((dim,), jnp.int32),
            pltpu.VMEM((dim,), jnp.float32),
            pltpu.VMEM((rows, dim), jnp.float32),
            pltpu.VMEM((rows, dim), jnp.float32),
            pltpu.VMEM((num_coeffs, rows, num_channels), jnp.float32),
            pltpu.VMEM((num_coeffs, rows, num_channels), jnp.float32),
            pltpu.SemaphoreType.DMA,
            pltpu.SemaphoreType.DMA,
            pltpu.SemaphoreType.DMA,
            pltpu.SemaphoreType.DMA,
        ],
        compiler_params=pltpu.CompilerParams(needs_layout_passes=False),
    )
    def packer(x_hbm, cidx_hbm, keep_hbm, out_hbm, cidx_v, keep_v,
               ib0, ib1, ob0, ob1, isem0, isem1, osem0, osem1):
        ibufs, obufs = (ib0, ib1), (ob0, ob1)
        isems, osems = (isem0, isem1), (osem0, osem1)
        wid = lax.axis_index("s") * num_cores + lax.axis_index("c")
        out_base = wid * rows_per_worker
        in_base = row_base + out_base

        def start_in(ci, b):
            src = x_hbm.at[pl.ds(in_base + ci * rows, rows), :]
            pltpu.async_copy(src, ibufs[b], isems[b])

        def wait_in(b):
            pltpu.make_async_copy(x_hbm.at[pl.ds(0, rows), :],
                                  ibufs[b], isems[b]).wait()

        def start_out(ci, b):
            ro = out_base + ci * rows
            for j in range(num_coeffs):
                pltpu.async_copy(obufs[b].at[j],
                                 out_hbm.at[j, pl.ds(ro, rows), :], osems[b])

        def wait_out(b):
            for j in range(num_coeffs):
                pltpu.make_async_copy(obufs[b].at[j],
                                      out_hbm.at[j, pl.ds(0, rows), :],
                                      osems[b]).wait()

        def compute(b):
            ib, ob = ibufs[b], obufs[b]

            @plsc.parallel_loop(0, rows)
            def _(r):
                rv = jnp.broadcast_to(r, (_LANES,)).astype(jnp.int32)
                for j in range(num_coeffs):
                    for g in range(num_channels // _LANES):
                        s = j * num_channels + g * _LANES
                        ci = cidx_v[pl.ds(s, _LANES)]
                        kv = keep_v[pl.ds(s, _LANES)]
                        vals = plsc.load_gather(ib, [rv, ci]) * kv
                        ob[j, r, pl.ds(g * _LANES, _LANES)] = vals

        pltpu.sync_copy(cidx_hbm, cidx_v)
        pltpu.sync_copy(keep_hbm, keep_v)
        start_in(0, 0)
        start_in(1, 1)

        @pl.loop(0, chunks, step=2)
        def _(i):
            for b in range(2):
                ci = i + b
                wait_in(b)

                @pl.when(ci >= 2)
                def _():
                    wait_out(b)

                compute(b)
                start_out(ci, b)

                @pl.when(ci + 2 < chunks)
                def _():
                    start_in(ci + 2, b)

        wait_out(0)
        wait_out(1)

    return packer


def _tc_body(x_ref, idx_ref, msk_ref, o_ref):
    blk, dim = x_ref.shape
    groups = dim // 128
    x = x_ref[...]
    idx = idx_ref[...]
    lidx = jnp.bitwise_and(idx, 127)
    sgrp = jnp.right_shift(idx, 7)
    for j in range(groups):
        lo, hi = j * 128, (j + 1) * 128
        idxb = jnp.broadcast_to(lidx[:, lo:hi], (blk, 128))
        sg = sgrp[:, lo:hi]
        acc = jnp.zeros((blk, 128), jnp.float32)
        for s in range(groups):
            g = jnp.take_along_axis(x[:, s * 128:(s + 1) * 128], idxb,
                                    axis=1, mode="promise_in_bounds")
            acc = jnp.where(sg == s, g, acc)
        o_ref[j, :, :] = jnp.where(msk_ref[0, j:j + 1, :] != 0,
                                   jnp.float32(0.0), acc)


def _make_tc_packer(n_rows, tc_rows, dim, num_coeffs, num_channels):
    blk = _TC_BLOCK
    return pl.pallas_call(
        _tc_body,
        grid=(tc_rows // blk,),
        in_specs=[
            pl.BlockSpec((blk, dim), lambda i: (i, 0)),
            pl.BlockSpec((1, dim), lambda i: (0, 0)),
            pl.BlockSpec((1, num_coeffs, num_channels), lambda i: (0, 0, 0)),
        ],
        out_specs=pl.BlockSpec((num_coeffs, blk, num_channels),
                               lambda i: (0, i, 0)),
        out_shape=jax.ShapeDtypeStruct((num_coeffs, n_rows, num_channels),
                                       jnp.float32),
    )


def kernel(x_flat, pack_index, pad_mask):
    n, dim = x_flat.shape
    num_coeffs, num_channels = pad_mask.shape[1], pad_mask.shape[2]
    info = plsc.get_sparse_core_info()
    num_workers = info.num_cores * info.num_subcores
    sc_rows = _SC_ROWS
    tc_rows = n - sc_rows
    assert tc_rows % _TC_BLOCK == 0

    pidx = pack_index.astype(jnp.int32)
    mask_flat = pad_mask.reshape(-1)
    cidx = jnp.where(mask_flat, 0, pidx)
    keep = 1.0 - mask_flat.astype(jnp.float32)

    sc_packer = _make_sc_packer(dim, tc_rows, sc_rows, num_workers,
                                info.num_cores, num_coeffs, num_channels)
    sc_out = sc_packer(x_flat, cidx, keep)

    tc_packer = _make_tc_packer(n, tc_rows, dim, num_coeffs, num_channels)
    tc_out = tc_packer(x_flat, pidx[None, :], pad_mask.astype(jnp.int32))

    planes = lax.dynamic_update_slice(tc_out, sc_out, (0, tc_rows, 0))
    return jnp.transpose(planes, (1, 0, 2))


# speedup vs baseline: 3.0392x; 1.2323x over previous
import functools

import jax
import jax.numpy as jnp
from jax import lax
from jax.experimental import pallas as pl
from jax.experimental.pallas import tpu as pltpu
from jax.experimental.pallas import tpu_sc as plsc

_LANES = 16
_ROWS_PER_CHUNK = 16
_SC_ROWS = 47104
_TC_BLOCK = 608


def _make_sc_packer(dim, row_base, sc_rows, num_workers, num_cores,
                    num_coeffs, num_channels):
    rows = _ROWS_PER_CHUNK
    rows_per_worker = sc_rows // num_workers
    chunks = rows_per_worker // rows
    assert chunks % 2 == 0 and chunks * rows == rows_per_worker

    mesh = plsc.VectorSubcoreMesh(core_axis_name="c", subcore_axis_name="s")

    @functools.partial(
        pl.kernel,
        mesh=mesh,
        out_type=jax.ShapeDtypeStruct((num_coeffs, sc_rows, num_channels),
                                      jnp.float32),
        scratch_types=[
            pltpu.VMEM((dim,), jnp.int32),
            pltpu.VMEM((dim,), jnp.float32),
            pltpu.VMEM((rows, dim), jnp.float32),
            pltpu.VMEM((rows, dim), jnp.float32),
            pltpu.VMEM((num_coeffs, rows, num_channels), jnp.float32),
            pltpu.VMEM((num_coeffs, rows, num_channels), jnp.float32),
            pltpu.SemaphoreType.DMA,
            pltpu.SemaphoreType.DMA,
            pltpu.SemaphoreType.DMA,
            pltpu.SemaphoreType.DMA,
        ],
        compiler_params=pltpu.CompilerParams(needs_layout_passes=False),
    )
    def packer(x_hbm, cidx_hbm, keep_hbm, out_hbm, cidx_v, keep_v,
               ib0, ib1, ob0, ob1, isem0, isem1, osem0, osem1):
        ibufs, obufs = (ib0, ib1), (ob0, ob1)
        isems, osems = (isem0, isem1), (osem0, osem1)
        wid = lax.axis_index("s") * num_cores + lax.axis_index("c")
        out_base = wid * rows_per_worker
        in_base = row_base + out_base

        def start_in(ci, b):
            src = x_hbm.at[pl.ds(in_base + ci * rows, rows), :]
            pltpu.async_copy(src, ibufs[b], isems[b])

        def wait_in(b):
            pltpu.make_async_copy(x_hbm.at[pl.ds(0, rows), :],
                                  ibufs[b], isems[b]).wait()

        def start_out(ci, b):
            ro = out_base + ci * rows
            for j in range(num_coeffs):
                pltpu.async_copy(obufs[b].at[j],
                                 out_hbm.at[j, pl.ds(ro, rows), :], osems[b])

        def wait_out(b):
            for j in range(num_coeffs):
                pltpu.make_async_copy(obufs[b].at[j],
                                      out_hbm.at[j, pl.ds(0, rows), :],
                                      osems[b]).wait()

        def compute(b):
            ib, ob = ibufs[b], obufs[b]

            @plsc.parallel_loop(0, rows)
            def _(r):
                rv = jnp.broadcast_to(r, (_LANES,)).astype(jnp.int32)
                for j in range(num_coeffs):
                    for g in range(num_channels // _LANES):
                        s = j * num_channels + g * _LANES
                        ci = cidx_v[pl.ds(s, _LANES)]
                        kv = keep_v[pl.ds(s, _LANES)]
                        vals = plsc.load_gather(ib, [rv, ci]) * kv
                        ob[j, r, pl.ds(g * _LANES, _LANES)] = vals

        pltpu.sync_copy(cidx_hbm, cidx_v)
        pltpu.sync_copy(keep_hbm, keep_v)
        start_in(0, 0)
        start_in(1, 1)

        @pl.loop(0, chunks, step=2)
        def _(i):
            for b in range(2):
                ci = i + b
                wait_in(b)

                @pl.when(ci >= 2)
                def _():
                    wait_out(b)

                compute(b)
                start_out(ci, b)

                @pl.when(ci + 2 < chunks)
                def _():
                    start_in(ci + 2, b)

        wait_out(0)
        wait_out(1)

    return packer


def _tc_body(x_ref, idx_ref, msk_ref, o_ref):
    blk, dim = x_ref.shape
    groups = dim // 128
    x = x_ref[...]
    idx = idx_ref[...]
    lidx = jnp.bitwise_and(idx, 127)
    sgrp = jnp.right_shift(idx, 7)
    for j in range(groups):
        lo, hi = j * 128, (j + 1) * 128
        idxb = jnp.broadcast_to(lidx[:, lo:hi], (blk, 128))
        sg = sgrp[:, lo:hi]
        acc = jnp.zeros((blk, 128), jnp.float32)
        for s in range(groups):
            g = jnp.take_along_axis(x[:, s * 128:(s + 1) * 128], idxb,
                                    axis=1, mode="promise_in_bounds")
            acc = jnp.where(sg == s, g, acc)
        o_ref[j, :, :] = jnp.where(msk_ref[0, j:j + 1, :] != 0,
                                   jnp.float32(0.0), acc)


def _make_tc_packer(n_rows, tc_rows, dim, num_coeffs, num_channels):
    blk = _TC_BLOCK
    return pl.pallas_call(
        _tc_body,
        grid=(tc_rows // blk,),
        in_specs=[
            pl.BlockSpec((blk, dim), lambda i: (i, 0)),
            pl.BlockSpec((1, dim), lambda i: (0, 0)),
            pl.BlockSpec((1, num_coeffs, num_channels), lambda i: (0, 0, 0)),
        ],
        out_specs=pl.BlockSpec((num_coeffs, blk, num_channels),
                               lambda i: (0, i, 0)),
        out_shape=jax.ShapeDtypeStruct((num_coeffs, n_rows, num_channels),
                                       jnp.float32),
    )


def kernel(x_flat, pack_index, pad_mask):
    n, dim = x_flat.shape
    num_coeffs, num_channels = pad_mask.shape[1], pad_mask.shape[2]
    info = plsc.get_sparse_core_info()
    num_workers = info.num_cores * info.num_subcores
    sc_rows = _SC_ROWS
    tc_rows = n - sc_rows
    assert tc_rows % _TC_BLOCK == 0

    pidx = pack_index.astype(jnp.int32)
    mask_flat = pad_mask.reshape(-1)
    cidx = jnp.where(mask_flat, 0, pidx)
    keep = 1.0 - mask_flat.astype(jnp.float32)

    sc_packer = _make_sc_packer(dim, tc_rows, sc_rows, num_workers,
                                info.num_cores, num_coeffs, num_channels)
    sc_out = sc_packer(x_flat, cidx, keep)

    tc_packer = _make_tc_packer(n, tc_rows, dim, num_coeffs, num_channels)
    tc_out = tc_packer(x_flat, pidx[None, :], pad_mask.astype(jnp.int32))

    planes = lax.dynamic_update_slice(tc_out, sc_out, (0, tc_rows, 0))
    return jnp.transpose(planes, (1, 0, 2))


# speedup vs baseline: 3.7214x; 1.2245x over previous
import functools

import jax
import jax.numpy as jnp
from jax import lax
from jax.experimental import pallas as pl
from jax.experimental.pallas import tpu as pltpu
from jax.experimental.pallas import tpu_sc as plsc

_LANES = 16
_ROWS_PER_CHUNK = 16
_SC_ROWS = 38912
_TC_BLOCK = 664


def _make_sc_packer(dim, row_base, sc_rows, num_workers, num_cores,
                    num_coeffs, num_channels):
    rows = _ROWS_PER_CHUNK
    rows_per_worker = sc_rows // num_workers
    chunks = rows_per_worker // rows
    assert chunks % 2 == 0 and chunks * rows == rows_per_worker

    mesh = plsc.VectorSubcoreMesh(core_axis_name="c", subcore_axis_name="s")

    @functools.partial(
        pl.kernel,
        mesh=mesh,
        out_type=jax.ShapeDtypeStruct((num_coeffs, sc_rows, num_channels),
                                      jnp.float32),
        scratch_types=[
            pltpu.VMEM((dim,), jnp.int32),
            pltpu.VMEM((dim,), jnp.float32),
            pltpu.VMEM((rows, dim), jnp.float32),
            pltpu.VMEM((rows, dim), jnp.float32),
            pltpu.VMEM((num_coeffs, rows, num_channels), jnp.float32),
            pltpu.VMEM((num_coeffs, rows, num_channels), jnp.float32),
            pltpu.SemaphoreType.DMA,
            pltpu.SemaphoreType.DMA,
            pltpu.SemaphoreType.DMA,
            pltpu.SemaphoreType.DMA,
        ],
        compiler_params=pltpu.CompilerParams(needs_layout_passes=False),
    )
    def packer(x_hbm, cidx_hbm, keep_hbm, out_hbm, cidx_v, keep_v,
               ib0, ib1, ob0, ob1, isem0, isem1, osem0, osem1):
        ibufs, obufs = (ib0, ib1), (ob0, ob1)
        isems, osems = (isem0, isem1), (osem0, osem1)
        wid = lax.axis_index("s") * num_cores + lax.axis_index("c")
        out_base = wid * rows_per_worker
        in_base = row_base + out_base

        def start_in(ci, b):
            src = x_hbm.at[pl.ds(in_base + ci * rows, rows), :]
            pltpu.async_copy(src, ibufs[b], isems[b])

        def wait_in(b):
            pltpu.make_async_copy(x_hbm.at[pl.ds(0, rows), :],
                                  ibufs[b], isems[b]).wait()

        def start_out(ci, b):
            ro = out_base + ci * rows
            for j in range(num_coeffs):
                pltpu.async_copy(obufs[b].at[j],
                                 out_hbm.at[j, pl.ds(ro, rows), :], osems[b])

        def wait_out(b):
            for j in range(num_coeffs):
                pltpu.make_async_copy(obufs[b].at[j],
                                      out_hbm.at[j, pl.ds(0, rows), :],
                                      osems[b]).wait()

        def compute(b):
            ib, ob = ibufs[b], obufs[b]

            @plsc.parallel_loop(0, rows)
            def _(r):
                rv = jnp.broadcast_to(r, (_LANES,)).astype(jnp.int32)
                for j in range(num_coeffs):
                    for g in range(num_channels // _LANES):
                        s = j * num_channels + g * _LANES
                        ci = cidx_v[pl.ds(s, _LANES)]
                        kv = keep_v[pl.ds(s, _LANES)]
                        vals = plsc.load_gather(ib, [rv, ci]) * kv
                        ob[j, r, pl.ds(g * _LANES, _LANES)] = vals

        pltpu.sync_copy(cidx_hbm, cidx_v)
        pltpu.sync_copy(keep_hbm, keep_v)
        start_in(0, 0)
        start_in(1, 1)

        @pl.loop(0, chunks, step=2)
        def _(i):
            for b in range(2):
                ci = i + b
                wait_in(b)

                @pl.when(ci >= 2)
                def _():
                    wait_out(b)

                compute(b)
                start_out(ci, b)

                @pl.when(ci + 2 < chunks)
                def _():
                    start_in(ci + 2, b)

        wait_out(0)
        wait_out(1)

    return packer


def _tc_body(x_ref, idx_ref, msk_ref, o_ref):
    blk, dim = x_ref.shape
    groups = dim // 128
    if groups == 9:
        srcs = {0: (0,), 1: (1, 2, 3), 2: (1, 2, 3), 3: (1, 2, 3)}
        for j in range(4, 9):
            srcs[j] = (4, 5, 6, 7, 8)
    else:
        srcs = {j: tuple(range(groups)) for j in range(groups)}
    x = x_ref[...]
    idx = idx_ref[...]
    lidx = jnp.bitwise_and(idx, 127)
    sgrp = jnp.right_shift(idx, 7)
    for j in range(groups):
        lo, hi = j * 128, (j + 1) * 128
        idxb = jnp.broadcast_to(lidx[:, lo:hi], (blk, 128))
        sg = sgrp[:, lo:hi]
        acc = jnp.zeros((blk, 128), jnp.float32)
        for s in srcs[j]:
            g = jnp.take_along_axis(x[:, s * 128:(s + 1) * 128], idxb,
                                    axis=1, mode="promise_in_bounds")
            acc = jnp.where(sg == s, g, acc)
        o_ref[j, :, :] = jnp.where(msk_ref[0, j:j + 1, :] != 0,
                                   jnp.float32(0.0), acc)


def _make_tc_packer(n_rows, tc_rows, dim, num_coeffs, num_channels):
    blk = _TC_BLOCK
    return pl.pallas_call(
        _tc_body,
        grid=(tc_rows // blk,),
        in_specs=[
            pl.BlockSpec((blk, dim), lambda i: (i, 0)),
            pl.BlockSpec((1, dim), lambda i: (0, 0)),
            pl.BlockSpec((1, num_coeffs, num_channels), lambda i: (0, 0, 0)),
        ],
        out_specs=pl.BlockSpec((num_coeffs, blk, num_channels),
                               lambda i: (0, i, 0)),
        out_shape=jax.ShapeDtypeStruct((num_coeffs, n_rows, num_channels),
                                       jnp.float32),
    )


def kernel(x_flat, pack_index, pad_mask):
    n, dim = x_flat.shape
    num_coeffs, num_channels = pad_mask.shape[1], pad_mask.shape[2]
    info = plsc.get_sparse_core_info()
    num_workers = info.num_cores * info.num_subcores
    sc_rows = _SC_ROWS
    tc_rows = n - sc_rows
    assert tc_rows % _TC_BLOCK == 0

    pidx = pack_index.astype(jnp.int32)
    mask_flat = pad_mask.reshape(-1)
    cidx = jnp.where(mask_flat, 0, pidx)
    keep = 1.0 - mask_flat.astype(jnp.float32)

    sc_packer = _make_sc_packer(dim, tc_rows, sc_rows, num_workers,
                                info.num_cores, num_coeffs, num_channels)
    sc_out = sc_packer(x_flat, cidx, keep)

    tc_packer = _make_tc_packer(n, tc_rows, dim, num_coeffs, num_channels)
    tc_out = tc_packer(x_flat, pidx[None, :], pad_mask.astype(jnp.int32))

    planes = lax.dynamic_update_slice(tc_out, sc_out, (0, tc_rows, 0))
    return jnp.transpose(planes, (1, 0, 2))


# speedup vs baseline: 4.7164x; 1.2674x over previous
import functools

import jax
import jax.numpy as jnp
from jax import lax
from jax.experimental import pallas as pl
from jax.experimental.pallas import tpu as pltpu
from jax.experimental.pallas import tpu_sc as plsc

_LANES = 16
_ROWS_PER_CHUNK = 16
_SC_ROWS = 29696
_TC_BLOCK = 1352


def _make_sc_packer(dim, row_base, sc_rows, num_workers, num_cores,
                    num_coeffs, num_channels):
    rows = _ROWS_PER_CHUNK
    rows_per_worker = sc_rows // num_workers
    chunks = rows_per_worker // rows
    assert chunks % 2 == 0 and chunks * rows == rows_per_worker

    mesh = plsc.VectorSubcoreMesh(core_axis_name="c", subcore_axis_name="s")

    @functools.partial(
        pl.kernel,
        mesh=mesh,
        out_type=jax.ShapeDtypeStruct((num_coeffs, sc_rows, num_channels),
                                      jnp.float32),
        scratch_types=[
            pltpu.VMEM((dim,), jnp.int32),
            pltpu.VMEM((dim,), jnp.float32),
            pltpu.VMEM((rows, dim), jnp.float32),
            pltpu.VMEM((rows, dim), jnp.float32),
            pltpu.VMEM((num_coeffs, rows, num_channels), jnp.float32),
            pltpu.VMEM((num_coeffs, rows, num_channels), jnp.float32),
            pltpu.SemaphoreType.DMA,
            pltpu.SemaphoreType.DMA,
            pltpu.SemaphoreType.DMA,
            pltpu.SemaphoreType.DMA,
        ],
        compiler_params=pltpu.CompilerParams(needs_layout_passes=False),
    )
    def packer(x_hbm, cidx_hbm, keep_hbm, out_hbm, cidx_v, keep_v,
               ib0, ib1, ob0, ob1, isem0, isem1, osem0, osem1):
        ibufs, obufs = (ib0, ib1), (ob0, ob1)
        isems, osems = (isem0, isem1), (osem0, osem1)
        wid = lax.axis_index("s") * num_cores + lax.axis_index("c")
        out_base = wid * rows_per_worker
        in_base = row_base + out_base

        def start_in(ci, b):
            src = x_hbm.at[pl.ds(in_base + ci * rows, rows), :]
            pltpu.async_copy(src, ibufs[b], isems[b])

        def wait_in(b):
            pltpu.make_async_copy(x_hbm.at[pl.ds(0, rows), :],
                                  ibufs[b], isems[b]).wait()

        def start_out(ci, b):
            ro = out_base + ci * rows
            for j in range(num_coeffs):
                pltpu.async_copy(obufs[b].at[j],
                                 out_hbm.at[j, pl.ds(ro, rows), :], osems[b])

        def wait_out(b):
            for j in range(num_coeffs):
                pltpu.make_async_copy(obufs[b].at[j],
                                      out_hbm.at[j, pl.ds(0, rows), :],
                                      osems[b]).wait()

        def compute(b):
            ib, ob = ibufs[b], obufs[b]

            @plsc.parallel_loop(0, rows)
            def _(r):
                rv = jnp.broadcast_to(r, (_LANES,)).astype(jnp.int32)
                for j in range(num_coeffs):
                    for g in range(num_channels // _LANES):
                        s = j * num_channels + g * _LANES
                        ci = cidx_v[pl.ds(s, _LANES)]
                        kv = keep_v[pl.ds(s, _LANES)]
                        vals = plsc.load_gather(ib, [rv, ci]) * kv
                        ob[j, r, pl.ds(g * _LANES, _LANES)] = vals

        pltpu.sync_copy(cidx_hbm, cidx_v)
        pltpu.sync_copy(keep_hbm, keep_v)
        start_in(0, 0)
        start_in(1, 1)

        @pl.loop(0, chunks, step=2)
        def _(i):
            for b in range(2):
                ci = i + b
                wait_in(b)

                @pl.when(ci >= 2)
                def _():
                    wait_out(b)

                compute(b)
                start_out(ci, b)

                @pl.when(ci + 2 < chunks)
                def _():
                    start_in(ci + 2, b)

        wait_out(0)
        wait_out(1)

    return packer


def _tc_body(x_ref, idx_ref, msk_ref, o_ref):
    blk, dim = x_ref.shape
    groups = dim // 128
    if groups == 9:
        srcs = {0: (0,), 1: (1, 2, 3), 2: (1, 2, 3), 3: (1, 2, 3)}
        for j in range(4, 9):
            srcs[j] = (4, 5, 6, 7, 8)
    else:
        srcs = {j: tuple(range(groups)) for j in range(groups)}
    x = x_ref[...]
    idx = idx_ref[...]
    lidx = jnp.bitwise_and(idx, 127)
    sgrp = jnp.right_shift(idx, 7)
    for j in range(groups):
        lo, hi = j * 128, (j + 1) * 128
        idxb = jnp.broadcast_to(lidx[:, lo:hi], (blk, 128))
        sg = sgrp[:, lo:hi]
        acc = jnp.zeros((blk, 128), jnp.float32)
        for s in srcs[j]:
            g = jnp.take_along_axis(x[:, s * 128:(s + 1) * 128], idxb,
                                    axis=1, mode="promise_in_bounds")
            acc = jnp.where(sg == s, g, acc)
        o_ref[j, :, :] = jnp.where(msk_ref[0, j:j + 1, :] != 0,
                                   jnp.float32(0.0), acc)


def _make_tc_packer(n_rows, tc_rows, dim, num_coeffs, num_channels):
    blk = _TC_BLOCK
    return pl.pallas_call(
        _tc_body,
        grid=(tc_rows // blk,),
        in_specs=[
            pl.BlockSpec((blk, dim), lambda i: (i, 0)),
            pl.BlockSpec((1, dim), lambda i: (0, 0)),
            pl.BlockSpec((1, num_coeffs, num_channels), lambda i: (0, 0, 0)),
        ],
        out_specs=pl.BlockSpec((num_coeffs, blk, num_channels),
                               lambda i: (0, i, 0)),
        out_shape=jax.ShapeDtypeStruct((num_coeffs, n_rows, num_channels),
                                       jnp.float32),
    )


def kernel(x_flat, pack_index, pad_mask):
    n, dim = x_flat.shape
    num_coeffs, num_channels = pad_mask.shape[1], pad_mask.shape[2]
    info = plsc.get_sparse_core_info()
    num_workers = info.num_cores * info.num_subcores
    sc_rows = _SC_ROWS
    tc_rows = n - sc_rows
    assert tc_rows % _TC_BLOCK == 0

    pidx = pack_index.astype(jnp.int32)
    mask_flat = pad_mask.reshape(-1)
    cidx = jnp.where(mask_flat, 0, pidx)
    keep = 1.0 - mask_flat.astype(jnp.float32)

    sc_packer = _make_sc_packer(dim, tc_rows, sc_rows, num_workers,
                                info.num_cores, num_coeffs, num_channels)
    sc_out = sc_packer(x_flat, cidx, keep)

    tc_packer = _make_tc_packer(n, tc_rows, dim, num_coeffs, num_channels)
    tc_out = tc_packer(x_flat, pidx[None, :], pad_mask.astype(jnp.int32))

    planes = lax.dynamic_update_slice(tc_out, sc_out, (0, tc_rows, 0))
    return jnp.transpose(planes, (1, 0, 2))


# speedup vs baseline: 5.0736x; 1.0757x over previous
import functools

import jax
import jax.numpy as jnp
from jax import lax
from jax.experimental import pallas as pl
from jax.experimental.pallas import tpu as pltpu
from jax.experimental.pallas import tpu_sc as plsc

_LANES = 16
_ROWS_PER_CHUNK = 24
_SC_ROWS = 27648
_TC_BLOCK = 952


def _make_sc_packer(dim, row_base, sc_rows, num_workers, num_cores,
                    num_coeffs, num_channels):
    rows = _ROWS_PER_CHUNK
    rows_per_worker = sc_rows // num_workers
    chunks = rows_per_worker // rows
    assert chunks % 2 == 0 and chunks * rows == rows_per_worker

    mesh = plsc.VectorSubcoreMesh(core_axis_name="c", subcore_axis_name="s")

    @functools.partial(
        pl.kernel,
        mesh=mesh,
        out_type=jax.ShapeDtypeStruct((num_coeffs, sc_rows, num_channels),
                                      jnp.float32),
        scratch_types=[
            pltpu.VMEM((dim,), jnp.int32),
            pltpu.VMEM((dim,), jnp.float32),
            pltpu.VMEM((rows, dim), jnp.float32),
            pltpu.VMEM((rows, dim), jnp.float32),
            pltpu.VMEM((num_coeffs, rows, num_channels), jnp.float32),
            pltpu.VMEM((num_coeffs, rows, num_channels), jnp.float32),
            pltpu.SemaphoreType.DMA,
            pltpu.SemaphoreType.DMA,
            pltpu.SemaphoreType.DMA,
            pltpu.SemaphoreType.DMA,
        ],
        compiler_params=pltpu.CompilerParams(needs_layout_passes=False),
    )
    def packer(x_hbm, cidx_hbm, keep_hbm, out_hbm, cidx_v, keep_v,
               ib0, ib1, ob0, ob1, isem0, isem1, osem0, osem1):
        ibufs, obufs = (ib0, ib1), (ob0, ob1)
        isems, osems = (isem0, isem1), (osem0, osem1)
        wid = lax.axis_index("s") * num_cores + lax.axis_index("c")
        out_base = wid * rows_per_worker
        in_base = row_base + out_base

        def start_in(ci, b):
            src = x_hbm.at[pl.ds(in_base + ci * rows, rows), :]
            pltpu.async_copy(src, ibufs[b], isems[b])

        def wait_in(b):
            pltpu.make_async_copy(x_hbm.at[pl.ds(0, rows), :],
                                  ibufs[b], isems[b]).wait()

        def start_out(ci, b):
            ro = out_base + ci * rows
            for j in range(num_coeffs):
                pltpu.async_copy(obufs[b].at[j],
                                 out_hbm.at[j, pl.ds(ro, rows), :], osems[b])

        def wait_out(b):
            for j in range(num_coeffs):
                pltpu.make_async_copy(obufs[b].at[j],
                                      out_hbm.at[j, pl.ds(0, rows), :],
                                      osems[b]).wait()

        def compute(b):
            ib, ob = ibufs[b], obufs[b]

            @plsc.parallel_loop(0, rows)
            def _(r):
                rv = jnp.broadcast_to(r, (_LANES,)).astype(jnp.int32)
                for j in range(num_coeffs):
                    for g in range(num_channels // _LANES):
                        s = j * num_channels + g * _LANES
                        ci = cidx_v[pl.ds(s, _LANES)]
                        kv = keep_v[pl.ds(s, _LANES)]
                        vals = plsc.load_gather(ib, [rv, ci]) * kv
                        ob[j, r, pl.ds(g * _LANES, _LANES)] = vals

        pltpu.sync_copy(cidx_hbm, cidx_v)
        pltpu.sync_copy(keep_hbm, keep_v)
        start_in(0, 0)
        start_in(1, 1)

        @pl.loop(0, chunks, step=2)
        def _(i):
            for b in range(2):
                ci = i + b
                wait_in(b)

                @pl.when(ci >= 2)
                def _():
                    wait_out(b)

                compute(b)
                start_out(ci, b)

                @pl.when(ci + 2 < chunks)
                def _():
                    start_in(ci + 2, b)

        wait_out(0)
        wait_out(1)

    return packer


def _tc_body(x_ref, idx_ref, msk_ref, o_ref):
    blk, dim = x_ref.shape
    groups = dim // 128
    if groups == 9:
        srcs = {0: (0,), 1: (1, 2, 3), 2: (1, 2, 3), 3: (1, 2, 3)}
        for j in range(4, 9):
            srcs[j] = (4, 5, 6, 7, 8)
    else:
        srcs = {j: tuple(range(groups)) for j in range(groups)}
    x = x_ref[...]
    idx = idx_ref[...]
    lidx = jnp.bitwise_and(idx, 127)
    sgrp = jnp.right_shift(idx, 7)
    for j in range(groups):
        lo, hi = j * 128, (j + 1) * 128
        idxb = jnp.broadcast_to(lidx[:, lo:hi], (blk, 128))
        sg = sgrp[:, lo:hi]
        acc = jnp.zeros((blk, 128), jnp.float32)
        for s in srcs[j]:
            g = jnp.take_along_axis(x[:, s * 128:(s + 1) * 128], idxb,
                                    axis=1, mode="promise_in_bounds")
            acc = jnp.where(sg == s, g, acc)
        o_ref[j, :, :] = jnp.where(msk_ref[0, j:j + 1, :] != 0,
                                   jnp.float32(0.0), acc)


def _make_tc_packer(n_rows, tc_rows, dim, num_coeffs, num_channels):
    blk = _TC_BLOCK
    return pl.pallas_call(
        _tc_body,
        grid=(tc_rows // blk,),
        in_specs=[
            pl.BlockSpec((blk, dim), lambda i: (i, 0)),
            pl.BlockSpec((1, dim), lambda i: (0, 0)),
            pl.BlockSpec((1, num_coeffs, num_channels), lambda i: (0, 0, 0)),
        ],
        out_specs=pl.BlockSpec((num_coeffs, blk, num_channels),
                               lambda i: (0, i, 0)),
        out_shape=jax.ShapeDtypeStruct((num_coeffs, n_rows, num_channels),
                                       jnp.float32),
    )


def kernel(x_flat, pack_index, pad_mask):
    n, dim = x_flat.shape
    num_coeffs, num_channels = pad_mask.shape[1], pad_mask.shape[2]
    info = plsc.get_sparse_core_info()
    num_workers = info.num_cores * info.num_subcores
    sc_rows = _SC_ROWS
    tc_rows = n - sc_rows
    assert tc_rows % _TC_BLOCK == 0

    pidx = pack_index.astype(jnp.int32)
    mask_flat = pad_mask.reshape(-1)
    cidx = jnp.where(mask_flat, 0, pidx)
    keep = 1.0 - mask_flat.astype(jnp.float32)

    sc_packer = _make_sc_packer(dim, tc_rows, sc_rows, num_workers,
                                info.num_cores, num_coeffs, num_channels)
    sc_out = sc_packer(x_flat, cidx, keep)

    tc_packer = _make_tc_packer(n, tc_rows, dim, num_coeffs, num_channels)
    tc_out = tc_packer(x_flat, pidx[None, :], pad_mask.astype(jnp.int32))

    planes = lax.dynamic_update_slice(tc_out, sc_out, (0, tc_rows, 0))
    return jnp.transpose(planes, (1, 0, 2))


# speedup vs baseline: 5.1537x; 1.0158x over previous
import functools

import jax
import jax.numpy as jnp
from jax import lax
from jax.experimental import pallas as pl
from jax.experimental.pallas import tpu as pltpu
from jax.experimental.pallas import tpu_sc as plsc

_LANES = 16
_ROWS_PER_CHUNK = 24
_SC_ROWS = 32256
_TC_BLOCK = 1168


def _make_sc_packer(dim, row_base, sc_rows, num_workers, num_cores,
                    num_coeffs, num_channels):
    rows = _ROWS_PER_CHUNK
    rows_per_worker = sc_rows // num_workers
    chunks = rows_per_worker // rows
    assert chunks % 2 == 0 and chunks * rows == rows_per_worker

    mesh = plsc.VectorSubcoreMesh(core_axis_name="c", subcore_axis_name="s")

    @functools.partial(
        pl.kernel,
        mesh=mesh,
        out_type=jax.ShapeDtypeStruct((num_coeffs, sc_rows, num_channels),
                                      jnp.float32),
        scratch_types=[
            pltpu.VMEM((dim,), jnp.int32),
            pltpu.VMEM((dim,), jnp.float32),
            pltpu.VMEM((rows, dim), jnp.float32),
            pltpu.VMEM((rows, dim), jnp.float32),
            pltpu.VMEM((num_coeffs, rows, num_channels), jnp.float32),
            pltpu.VMEM((num_coeffs, rows, num_channels), jnp.float32),
            pltpu.SemaphoreType.DMA,
            pltpu.SemaphoreType.DMA,
            pltpu.SemaphoreType.DMA,
            pltpu.SemaphoreType.DMA,
        ],
        compiler_params=pltpu.CompilerParams(needs_layout_passes=False),
    )
    def packer(x_hbm, cidx_hbm, keep_hbm, out_hbm, cidx_v, keep_v,
               ib0, ib1, ob0, ob1, isem0, isem1, osem0, osem1):
        ibufs, obufs = (ib0, ib1), (ob0, ob1)
        isems, osems = (isem0, isem1), (osem0, osem1)
        wid = lax.axis_index("s") * num_cores + lax.axis_index("c")
        out_base = wid * rows_per_worker
        in_base = row_base + out_base

        def start_in(ci, b):
            src = x_hbm.at[pl.ds(in_base + ci * rows, rows), :]
            pltpu.async_copy(src, ibufs[b], isems[b])

        def wait_in(b):
            pltpu.make_async_copy(x_hbm.at[pl.ds(0, rows), :],
                                  ibufs[b], isems[b]).wait()

        def start_out(ci, b):
            ro = out_base + ci * rows
            for j in range(num_coeffs):
                pltpu.async_copy(obufs[b].at[j],
                                 out_hbm.at[j, pl.ds(ro, rows), :], osems[b])

        def wait_out(b):
            for j in range(num_coeffs):
                pltpu.make_async_copy(obufs[b].at[j],
                                      out_hbm.at[j, pl.ds(0, rows), :],
                                      osems[b]).wait()

        def compute(b):
            ib, ob = ibufs[b], obufs[b]

            @plsc.parallel_loop(0, rows)
            def _(r):
                rv = jnp.broadcast_to(r, (_LANES,)).astype(jnp.int32)
                for j in range(num_coeffs):
                    for g in range(num_channels // _LANES):
                        s = j * num_channels + g * _LANES
                        ci = cidx_v[pl.ds(s, _LANES)]
                        kv = keep_v[pl.ds(s, _LANES)]
                        vals = plsc.load_gather(ib, [rv, ci]) * kv
                        ob[j, r, pl.ds(g * _LANES, _LANES)] = vals

        pltpu.sync_copy(cidx_hbm, cidx_v)
        pltpu.sync_copy(keep_hbm, keep_v)
        start_in(0, 0)
        start_in(1, 1)

        @pl.loop(0, chunks, step=2)
        def _(i):
            for b in range(2):
                ci = i + b
                wait_in(b)

                @pl.when(ci >= 2)
                def _():
                    wait_out(b)

                compute(b)
                start_out(ci, b)

                @pl.when(ci + 2 < chunks)
                def _():
                    start_in(ci + 2, b)

        wait_out(0)
        wait_out(1)

    return packer


def _tc_body(x_ref, idx_ref, msk_ref, o_ref):
    blk, dim = x_ref.shape
    groups = dim // 128
    if groups == 9:
        srcs = {0: (0,), 1: (1, 2, 3), 2: (1, 2, 3), 3: (1, 2, 3)}
        for j in range(4, 9):
            srcs[j] = (4, 5, 6, 7, 8)
    else:
        srcs = {j: tuple(range(groups)) for j in range(groups)}
    x = x_ref[...]
    idx = idx_ref[...]
    lidx = jnp.bitwise_and(idx, 127)
    sgrp = jnp.right_shift(idx, 7)
    for j in range(groups):
        lo, hi = j * 128, (j + 1) * 128
        idxb = jnp.broadcast_to(lidx[:, lo:hi], (blk, 128))
        sg = sgrp[:, lo:hi]
        acc = jnp.zeros((blk, 128), jnp.float32)
        for s in srcs[j]:
            g = jnp.take_along_axis(x[:, s * 128:(s + 1) * 128], idxb,
                                    axis=1, mode="promise_in_bounds")
            acc = jnp.where(sg == s, g, acc)
        o_ref[j, :, :] = jnp.where(msk_ref[0, j:j + 1, :] != 0,
                                   jnp.float32(0.0), acc)


def _make_tc_packer(n_rows, tc_rows, dim, num_coeffs, num_channels):
    blk = _TC_BLOCK
    return pl.pallas_call(
        _tc_body,
        grid=(tc_rows // blk,),
        in_specs=[
            pl.BlockSpec((blk, dim), lambda i: (i, 0)),
            pl.BlockSpec((1, dim), lambda i: (0, 0)),
            pl.BlockSpec((1, num_coeffs, num_channels), lambda i: (0, 0, 0)),
        ],
        out_specs=pl.BlockSpec((num_coeffs, blk, num_channels),
                               lambda i: (0, i, 0)),
        out_shape=jax.ShapeDtypeStruct((num_coeffs, n_rows, num_channels),
                                       jnp.float32),
    )


def kernel(x_flat, pack_index, pad_mask):
    n, dim = x_flat.shape
    num_coeffs, num_channels = pad_mask.shape[1], pad_mask.shape[2]
    info = plsc.get_sparse_core_info()
    num_workers = info.num_cores * info.num_subcores
    sc_rows = _SC_ROWS
    tc_rows = n - sc_rows
    assert tc_rows % _TC_BLOCK == 0

    pidx = pack_index.astype(jnp.int32)
    mask_flat = pad_mask.reshape(-1)
    cidx = jnp.where(mask_flat, 0, pidx)
    keep = 1.0 - mask_flat.astype(jnp.float32)

    sc_packer = _make_sc_packer(dim, tc_rows, sc_rows, num_workers,
                                info.num_cores, num_coeffs, num_channels)
    sc_out = sc_packer(x_flat, cidx, keep)

    tc_packer = _make_tc_packer(n, tc_rows, dim, num_coeffs, num_channels)
    tc_out = tc_packer(x_flat, pidx[None, :], pad_mask.astype(jnp.int32))

    planes = lax.dynamic_update_slice(tc_out, sc_out, (0, tc_rows, 0))
    return jnp.transpose(planes, (1, 0, 2))
